# Initial kernel scaffold; baseline (speedup 1.0000x reference)
import jax
import jax.numpy as jnp
from jax.experimental import pallas as pl


def kernel(data, index, W):
    raise NotImplementedError("write your pallas kernel here")


# speedup vs baseline: 3.9076x; 3.9076x over previous
import functools

import jax
import jax.numpy as jnp
from jax import lax
from jax.experimental import pallas as pl
from jax.experimental.pallas import tpu as pltpu
from jax.experimental.pallas import tpu_sc as plsc

N_NODES = 10000
N_EDGES = 160000
D_FEAT = 128
UNITS = 2
EPS = 1e-07

NCHUNK = 8
CF = 16
NCH = 8
ROW = NCH * CF

NSUB = 16
NCORE = 2
EDGES_PER_SUB = N_EDGES // NSUB
EPB = 80
NBLK = EDGES_PER_SUB // EPB
NPAD = 10240
NODES_PER_SUB = NPAD // NSUB
ZROWS = 128

EB1 = 2000
NB3 = 1280


def _stage1_body(x_ref, o_ref):
    x = jnp.clip(x_ref[...], EPS, 1.0 - EPS)
    l1 = jnp.log(x)
    l2 = jnp.log(1.0 - x)
    for c in range(NCHUNK):
        o_ref[c, :, 0:CF] = l1[:, c * CF:(c + 1) * CF]
        o_ref[c, :, CF:2 * CF] = l2[:, c * CF:(c + 1) * CF]


def _stage1(data):
    return pl.pallas_call(
        _stage1_body,
        grid=(N_EDGES // EB1,),
        in_specs=[pl.BlockSpec((EB1, D_FEAT), lambda i: (i, 0))],
        out_specs=pl.BlockSpec((NCHUNK, EB1, 2 * CF), lambda i: (0, i, 0)),
        out_shape=jax.ShapeDtypeStruct((NCHUNK, N_EDGES, 2 * CF), jnp.float32),
    )(data)


def _sc_body(l_hbm, idx_hbm, p_hbm, out_hbm,
             l_buf, idx_buf, rows_buf, p_buf, zero_buf, acc):
    cid = lax.axis_index("c")
    sid = lax.axis_index("s")

    pltpu.sync_copy(p_hbm, p_buf)

    z16 = jnp.zeros((16,), jnp.float32)

    def zfill(i, carry):
        for t in range(ROW // 16):
            zero_buf[i, pl.ds(t * 16, 16)] = z16
        return carry

    lax.fori_loop(0, ZROWS, zfill, 0)

    for r in range(NCHUNK // NCORE):
        chunk = r * NCORE + cid

        for z in range(NODES_PER_SUB // ZROWS):
            pltpu.sync_copy(
                zero_buf,
                acc.at[pl.ds(sid * NODES_PER_SUB + z * ZROWS, ZROWS)])
        plsc.subcore_barrier()

        def blk(b, carry):
            base = sid * EDGES_PER_SUB + b * EPB
            pltpu.sync_copy(idx_hbm.at[pl.ds(base, EPB)], idx_buf)
            pltpu.sync_copy(l_hbm.at[chunk, pl.ds(base, EPB)], l_buf)

            def edge(j, c2):
                l1 = l_buf[j, 0:CF]
                l2 = l_buf[j, CF:2 * CF]
                for ch in range(NCH):
                    lv = l1 if ((ch // 2) % 2 == 0) else l2
                    rows_buf[j, pl.ds(ch * CF, CF)] = jnp.exp(p_buf[ch] * lv)
                return c2

            lax.fori_loop(0, EPB, edge, 0)
            pltpu.sync_copy(rows_buf, acc.at[idx_buf], add=True)
            return carry

        lax.fori_loop(0, NBLK, blk, 0)
        plsc.subcore_barrier()

        pltpu.sync_copy(
            acc.at[pl.ds(sid * NODES_PER_SUB, NODES_PER_SUB)],
            out_hbm.at[chunk, pl.ds(sid * NODES_PER_SUB, NODES_PER_SUB)])
        if r != NCHUNK // NCORE - 1:
            plsc.subcore_barrier()


def _sc_scatter(l_arr, index, p_arr):
    mesh = plsc.VectorSubcoreMesh(core_axis_name="c", subcore_axis_name="s")
    f = pl.kernel(
        _sc_body,
        out_type=jax.ShapeDtypeStruct((NCHUNK, NPAD, ROW), jnp.float32),
        mesh=mesh,
        scratch_types=[
            pltpu.VMEM((EPB, 2 * CF), jnp.float32),
            pltpu.VMEM((EPB,), jnp.int32),
            pltpu.VMEM((EPB, ROW), jnp.float32),
            pltpu.VMEM((NCH, 16), jnp.float32),
            pltpu.VMEM((ZROWS, ROW), jnp.float32),
            pltpu.VMEM_SHARED((NPAD, ROW), jnp.float32),
        ],
    )
    return f(l_arr, index, p_arr)


def _stage3_body(s_ref, qb_ref, o_ref):
    for c in range(NCHUNK):
        terms = []
        for ch in range(NCH):
            s = jnp.clip(s_ref[c, :, ch * CF:(ch + 1) * CF], EPS, None)
            q = qb_ref[0, ch:ch + 1, 0:CF]
            ab = qb_ref[1, ch:ch + 1, 0:CF]
            terms.append(jnp.exp(q * jnp.log(s)) * ab)
        for u in range(UNITS):
            num = terms[u] + terms[2 + u]
            den = terms[4 + u] + terms[6 + u]
            mult = 2.0 * jnp.clip(jnp.sign(den), 0.0, None) - 1.0
            den = jnp.where((den < EPS) & (den > -EPS), mult * EPS, den)
            o_ref[u, :, c * CF:(c + 1) * CF] = num / den


def _stage3(s_arr, qb_arr):
    return pl.pallas_call(
        _stage3_body,
        grid=(NPAD // NB3,),
        in_specs=[
            pl.BlockSpec((NCHUNK, NB3, ROW), lambda nb: (0, nb, 0)),
            pl.BlockSpec((2, NCH, 128), lambda nb: (0, 0, 0)),
        ],
        out_specs=pl.BlockSpec((UNITS, NB3, D_FEAT), lambda nb: (0, nb, 0)),
        out_shape=jax.ShapeDtypeStruct((UNITS, NPAD, D_FEAT), jnp.float32),
    )(s_arr, qb_arr)


def kernel(data, index, W):
    p = jax.nn.relu(W[0:4]).reshape(NCH)
    p_arr = jnp.broadcast_to(p[:, None], (NCH, 16))
    q = jax.nn.relu(W[4:8]).reshape(NCH)
    ab = W[8:12].reshape(NCH)
    qb_arr = jnp.stack([
        jnp.broadcast_to(q[:, None], (NCH, 128)),
        jnp.broadcast_to(ab[:, None], (NCH, 128)),
    ])
    l_arr = _stage1(data)
    s_arr = _sc_scatter(l_arr, index, p_arr)
    r_arr = _stage3(s_arr, qb_arr)
    return jnp.transpose(r_arr, (1, 2, 0))[:N_NODES]


# speedup vs baseline: 18.8536x; 4.8249x over previous
import functools

import jax
import jax.numpy as jnp
from jax import lax
from jax.experimental import pallas as pl
from jax.experimental.pallas import tpu as pltpu
from jax.experimental.pallas import tpu_sc as plsc

N_NODES = 10000
N_EDGES = 160000
D_FEAT = 128
UNITS = 2
EPS = 1e-07

NCHUNK = 8
CF = 16
NCH = 8
ROW = NCH * CF

NSUB = 16
NCORE = 2
EDGES_PER_SUB = N_EDGES // NSUB
EPB = 80
NBLK = EDGES_PER_SUB // EPB
NPAD = 10240
NODES_PER_SUB = NPAD // NSUB
ZROWS = 64

EB1 = 2000
NB3 = 1280


def _stage1_body(x_ref, o_ref):
    x = jnp.clip(x_ref[...], EPS, 1.0 - EPS)
    l1 = jnp.log(x)
    l2 = jnp.log(1.0 - x)
    for c in range(NCHUNK):
        o_ref[c, :, 0:CF] = l1[:, c * CF:(c + 1) * CF]
        o_ref[c, :, CF:2 * CF] = l2[:, c * CF:(c + 1) * CF]


def _stage1(data):
    return pl.pallas_call(
        _stage1_body,
        grid=(N_EDGES // EB1,),
        in_specs=[pl.BlockSpec((EB1, D_FEAT), lambda i: (i, 0))],
        out_specs=pl.BlockSpec((NCHUNK, EB1, 2 * CF), lambda i: (0, i, 0)),
        out_shape=jax.ShapeDtypeStruct((NCHUNK, N_EDGES, 2 * CF), jnp.float32),
    )(data)


ZCOPY = 8


def _sc_body(l_hbm, idx_hbm, p_hbm, out_hbm,
             l_buf, idx3, rows2, p_buf, acc,
             sem_in, sem_out):
    cid = lax.axis_index("c")
    sid = lax.axis_index("s")

    pltpu.sync_copy(p_hbm, p_buf)
    pv = [p_buf[ch] for ch in range(NCH)]

    z16 = jnp.zeros((16,), jnp.float32)

    for r in range(NCHUNK // NCORE):
        chunk = r * NCORE + cid

        def in_l(blk, par):
            base = sid * EDGES_PER_SUB + blk * EPB
            return pltpu.make_async_copy(
                l_hbm.at[chunk, pl.ds(base, EPB)], l_buf.at[par], sem_in)

        def in_i(blk, par3):
            return pltpu.make_async_copy(
                idx_hbm.at[sid, blk], idx3.at[par3], sem_in)

        def zfill(i, carry):
            for t in range(ROW // 16):
                rows2[0, i, pl.ds(t * 16, 16)] = z16
            return carry

        lax.fori_loop(0, EPB, zfill, 0)
        zd = [pltpu.async_copy(
                  rows2.at[0],
                  acc.at[pl.ds(sid * NODES_PER_SUB + z * EPB, EPB)],
                  sem_in)
              for z in range(ZCOPY)]
        for d in zd:
            d.wait()
        plsc.subcore_barrier()

        in_l(0, 0).start()
        in_i(0, 0).start()

        def blk_body(k, carry):
            p2 = lax.rem(k, 2)
            p3 = lax.rem(k, 3)
            in_l(k, p2).wait()
            in_i(k, p3).wait()

            @pl.when(k < NBLK - 1)
            def _():
                in_l(k + 1, 1 - p2).start()
                in_i(k + 1, lax.rem(k + 1, 3)).start()

            @plsc.parallel_loop(0, EPB, unroll=2)
            def edge(j):
                l1 = l_buf[p2, j, 0:CF]
                l2 = l_buf[p2, j, CF:2 * CF]
                for ch in range(NCH):
                    lv = l1 if ((ch // 2) % 2 == 0) else l2
                    rows2[p2, j, pl.ds(ch * CF, CF)] = jnp.exp(pv[ch] * lv)

            @pl.when(k > 0)
            def _():
                pltpu.make_async_copy(
                    rows2.at[1 - p2], acc.at[idx3.at[lax.rem(k + 2, 3)]],
                    sem_out).wait()

            pltpu.async_copy(
                rows2.at[p2], acc.at[idx3.at[p3]], sem_out, add=True)
            return carry

        lax.fori_loop(0, NBLK, blk_body, 0)
        pltpu.make_async_copy(
            rows2.at[(NBLK - 1) % 2], acc.at[idx3.at[(NBLK - 1) % 3]],
            sem_out).wait()
        plsc.subcore_barrier()

        pltpu.sync_copy(
            acc.at[pl.ds(sid * NODES_PER_SUB, NODES_PER_SUB)],
            out_hbm.at[chunk, pl.ds(sid * NODES_PER_SUB, NODES_PER_SUB)])
        if r != NCHUNK // NCORE - 1:
            plsc.subcore_barrier()


def _sc_scatter(l_arr, index, p_arr):
    mesh = plsc.VectorSubcoreMesh(core_axis_name="c", subcore_axis_name="s")
    f = pl.kernel(
        _sc_body,
        out_type=jax.ShapeDtypeStruct((NCHUNK, NPAD, ROW), jnp.float32),
        mesh=mesh,
        scratch_types=[
            pltpu.VMEM((2, EPB, 2 * CF), jnp.float32),
            pltpu.VMEM((3, EPB), jnp.int32),
            pltpu.VMEM((2, EPB, ROW), jnp.float32),
            pltpu.VMEM((NCH, 16), jnp.float32),
            pltpu.VMEM_SHARED((NPAD, ROW), jnp.float32),
            pltpu.SemaphoreType.DMA,
            pltpu.SemaphoreType.DMA,
        ],
    )
    return f(l_arr, index.reshape(NSUB, NBLK, EPB), p_arr)


def _stage3_body(s_ref, qb_ref, o_ref):
    for c in range(NCHUNK):
        terms = []
        for ch in range(NCH):
            s = jnp.clip(s_ref[c, :, ch * CF:(ch + 1) * CF], EPS, None)
            q = qb_ref[0, ch:ch + 1, 0:CF]
            ab = qb_ref[1, ch:ch + 1, 0:CF]
            terms.append(jnp.exp(q * jnp.log(s)) * ab)
        for u in range(UNITS):
            num = terms[u] + terms[2 + u]
            den = terms[4 + u] + terms[6 + u]
            mult = 2.0 * jnp.clip(jnp.sign(den), 0.0, None) - 1.0
            den = jnp.where((den < EPS) & (den > -EPS), mult * EPS, den)
            o_ref[u, :, c * CF:(c + 1) * CF] = num / den


def _stage3(s_arr, qb_arr):
    return pl.pallas_call(
        _stage3_body,
        grid=(NPAD // NB3,),
        in_specs=[
            pl.BlockSpec((NCHUNK, NB3, ROW), lambda nb: (0, nb, 0)),
            pl.BlockSpec((2, NCH, 128), lambda nb: (0, 0, 0)),
        ],
        out_specs=pl.BlockSpec((UNITS, NB3, D_FEAT), lambda nb: (0, nb, 0)),
        out_shape=jax.ShapeDtypeStruct((UNITS, NPAD, D_FEAT), jnp.float32),
    )(s_arr, qb_arr)


def kernel(data, index, W):
    p = jax.nn.relu(W[0:4]).reshape(NCH)
    p_arr = jnp.broadcast_to(p[:, None], (NCH, 16))
    q = jax.nn.relu(W[4:8]).reshape(NCH)
    ab = W[8:12].reshape(NCH)
    qb_arr = jnp.stack([
        jnp.broadcast_to(q[:, None], (NCH, 128)),
        jnp.broadcast_to(ab[:, None], (NCH, 128)),
    ])
    l_arr = _stage1(data)
    s_arr = _sc_scatter(l_arr, index, p_arr)
    r_arr = _stage3(s_arr, qb_arr)
    return jnp.transpose(r_arr, (1, 2, 0))[:N_NODES]
